# baseline (speedup 1.0000x reference)
import functools

import jax
import jax.numpy as jnp
from jax import lax
from jax.experimental import pallas as pl
from jax.experimental.pallas import tpu as pltpu
from jax.experimental.pallas import tpu_sc as plsc

N = 10000
NP = 10240
E = 320000
C = 128
DIM = 128

NC = 2
NS = 16
L = 16

K = 80
RB = NP // NS
G = NC * NS
CH_REAL = E // (G * K)
NCH = 128
STG = NCH // 4

_mesh = plsc.VectorSubcoreMesh(core_axis_name="c", subcore_axis_name="s")
_params = pltpu.CompilerParams(needs_layout_passes=False)


def _splat(buf, r):
    return plsc.load_gather(buf, [jnp.full((L,), r, dtype=jnp.int32)])


def _sc_degree(dst2, ew2):
    @functools.partial(
        pl.kernel,
        out_type=[
            jax.ShapeDtypeStruct((NP,), jnp.float32),
            jax.ShapeDtypeStruct((NP,), jnp.float32),
        ],
        mesh=_mesh,
        compiler_params=_params,
        scratch_types=[
            pltpu.VMEM((NCH, K), jnp.int32),
            pltpu.VMEM((NCH, K), jnp.float32),
            pltpu.VMEM((RB,), jnp.float32),
            pltpu.VMEM_SHARED((NP,), jnp.float32),
        ],
    )
    def body(dst_hbm, ew_hbm, deg0_hbm, deg1_hbm, dst_all, ew_all,
             obuf, deg_sh):
        c = lax.axis_index("c")
        s = lax.axis_index("s")
        rbase = s * RB
        cbase = (s * NC + c) * NCH

        seed = lax.broadcast(
            jnp.where(c == 0, jnp.float32(1.0), jnp.float32(0.0)), (L,))
        for t in range(RB // L):
            obuf[pl.ds(t * L, L)] = seed
        pltpu.sync_copy(obuf, deg_sh.at[pl.ds(rbase, RB)])

        pltpu.sync_copy(dst_hbm.at[pl.ds(cbase, NCH), :], dst_all)
        pltpu.sync_copy(ew_hbm.at[pl.ds(cbase, NCH), :], ew_all)
        plsc.subcore_barrier()

        def scat(j, carry):
            pltpu.sync_copy(ew_all.at[j], deg_sh.at[dst_all.at[j]], add=True)
            return carry
        lax.fori_loop(0, NCH, scat, 0)
        plsc.subcore_barrier()

        @pl.when(c == 0)
        def _():
            pltpu.sync_copy(deg_sh.at[pl.ds(rbase, RB)],
                            deg0_hbm.at[pl.ds(rbase, RB)])

        @pl.when(c == 1)
        def _():
            pltpu.sync_copy(deg_sh.at[pl.ds(rbase, RB)],
                            deg1_hbm.at[pl.ds(rbase, RB)])

    return body(dst2, ew2)


def _sc_aggregate(src2, dst2, ew2, h, dinv):
    @functools.partial(
        pl.kernel,
        out_type=[
            jax.ShapeDtypeStruct((N, DIM), jnp.float32),
            jax.ShapeDtypeStruct((N, DIM), jnp.float32),
        ],
        mesh=_mesh,
        compiler_params=_params,
        scratch_types=[
            pltpu.VMEM((NP,), jnp.float32),
            pltpu.VMEM((STG, K), jnp.int32),
            pltpu.VMEM((STG, K), jnp.int32),
            pltpu.VMEM((STG, K), jnp.float32),
            pltpu.VMEM((C,), jnp.float32),
            pltpu.VMEM((K, DIM), jnp.float32),
            pltpu.VMEM((K, DIM), jnp.float32),
            pltpu.VMEM((C,), jnp.float32),
            pltpu.VMEM_SHARED((N, DIM), jnp.float32),
            pltpu.SemaphoreType.DMA,
            pltpu.SemaphoreType.DMA,
            pltpu.SemaphoreType.DMA,
            pltpu.SemaphoreType.DMA,
        ],
    )
    def body(src_hbm, dst_hbm, ew_hbm, h_hbm, dinv_hbm, agg0_hbm, agg1_hbm,
             dinv_v, src_all, dst_all, ew_all, nbuf, rows_a, rows_b,
             dbuf, agg_sh, gsem_a1, gsem_a2, gsem_b1, gsem_b2):
        c = lax.axis_index("c")
        s = lax.axis_index("s")
        rbase = s * RB
        cbase = (s * NC + c) * NCH
        n16 = jnp.where(s == NS - 1, (N - (NS - 1) * RB) // L, RB // L)

        pltpu.sync_copy(dinv_hbm, dinv_v)

        zrow = rows_b.at[pl.ds(0, L), :]
        hrow = rows_a.at[pl.ds(0, L), :]
        for i in range(L):
            for j in range(DIM // L):
                rows_b[i, pl.ds(j * L, L)] = jnp.zeros((L,), jnp.float32)

        @pl.when(c == 1)
        def _():
            def zinit(t, carry):
                pltpu.sync_copy(zrow, agg_sh.at[pl.ds(rbase + t * L, L), :])
                return carry
            lax.fori_loop(0, n16, zinit, 0)

        @pl.when(c == 0)
        def _():
            def sinit(t, carry):
                rb = rbase + t * L
                pltpu.sync_copy(h_hbm.at[pl.ds(rb, L), :], hrow)
                v = dinv_v[pl.ds(rb, L)]
                dbuf[pl.ds(0, L)] = v * v
                for i in range(L):
                    sp = _splat(dbuf, i)
                    for j in range(DIM // L):
                        rows_a[i, pl.ds(j * L, L)] = (
                            rows_a[i, pl.ds(j * L, L)] * sp)
                pltpu.sync_copy(hrow, agg_sh.at[pl.ds(rb, L), :])
                return carry
            lax.fori_loop(0, n16, sinit, 0)
        plsc.subcore_barrier()

        def scale(j, rows):
            for g in range(K // L):
                vs = src_all[j, pl.ds(g * L, L)]
                vd = dst_all[j, pl.ds(g * L, L)]
                vw = ew_all[j, pl.ds(g * L, L)]
                a = plsc.load_gather(dinv_v, [vs])
                bn = plsc.load_gather(dinv_v, [vd])
                nbuf[pl.ds(g * L, L)] = a * vw * bn
            for r in range(K):
                sp = _splat(nbuf, r)
                for jj in range(DIM // L):
                    rows[r, pl.ds(jj * L, L)] = rows[r, pl.ds(jj * L, L)] * sp

        K2 = K // 2

        def gissue(j, rows, g1, g2):
            pltpu.async_copy(h_hbm.at[src_all.at[j, pl.ds(0, K2)]],
                             rows.at[pl.ds(0, K2), :], g1)
            pltpu.async_copy(h_hbm.at[src_all.at[j, pl.ds(K2, K2)]],
                             rows.at[pl.ds(K2, K2), :], g2)

        def gwait(j, rows, g1, g2):
            pltpu.make_async_copy(h_hbm.at[src_all.at[j, pl.ds(0, K2)]],
                                  rows.at[pl.ds(0, K2), :], g1).wait()
            pltpu.make_async_copy(h_hbm.at[src_all.at[j, pl.ds(K2, K2)]],
                                  rows.at[pl.ds(K2, K2), :], g2).wait()

        def step(j, rows, g1, g2, rows_o, go1, go2):
            gwait(j, rows, g1, g2)

            @pl.when(j + 1 < STG)
            def _():
                gissue(j + 1, rows_o, go1, go2)

            scale(j, rows)
            pltpu.sync_copy(rows, agg_sh.at[dst_all.at[j]], add=True)

        def msg_step(j, carry):
            @pl.when(j % 2 == 0)
            def _():
                step(j, rows_a, gsem_a1, gsem_a2, rows_b, gsem_b1, gsem_b2)

            @pl.when(j % 2 == 1)
            def _():
                step(j, rows_b, gsem_b1, gsem_b2, rows_a, gsem_a1, gsem_a2)
            return carry

        def stage_step(hf, carry):
            hb = cbase + hf * STG
            pltpu.sync_copy(src_hbm.at[pl.ds(hb, STG), :], src_all)
            pltpu.sync_copy(dst_hbm.at[pl.ds(hb, STG), :], dst_all)
            pltpu.sync_copy(ew_hbm.at[pl.ds(hb, STG), :], ew_all)
            gissue(0, rows_a, gsem_a1, gsem_a2)
            lax.fori_loop(0, STG, msg_step, 0)
            return carry
        lax.fori_loop(0, NCH // STG, stage_step, 0)
        plsc.subcore_barrier()

        @pl.when(s < NS - 1)
        def _():
            @pl.when(c == 0)
            def _():
                pltpu.sync_copy(agg_sh.at[pl.ds(rbase, RB), :],
                                agg0_hbm.at[pl.ds(rbase, RB), :])

            @pl.when(c == 1)
            def _():
                pltpu.sync_copy(agg_sh.at[pl.ds(rbase, RB), :],
                                agg1_hbm.at[pl.ds(rbase, RB), :])

        @pl.when(s == NS - 1)
        def _():
            def dump(t, carry):
                rb = rbase + t * L

                @pl.when(c == 0)
                def _():
                    pltpu.sync_copy(agg_sh.at[pl.ds(rb, L), :],
                                    agg0_hbm.at[pl.ds(rb, L), :])

                @pl.when(c == 1)
                def _():
                    pltpu.sync_copy(agg_sh.at[pl.ds(rb, L), :],
                                    agg1_hbm.at[pl.ds(rb, L), :])
                return carry
            lax.fori_loop(0, n16, dump, 0)

    return body(src2, dst2, ew2, h, dinv)


def _mm_body(x_ref, w_ref, d0_ref, d1_ref, h_ref, dinv_ref):
    h_ref[...] = jnp.dot(x_ref[...], w_ref[...],
                         preferred_element_type=jnp.float32)
    dinv_ref[...] = lax.rsqrt(d0_ref[...] + d1_ref[...])


def _matmul_dinv(x, w, deg0, deg1):
    blk = 1000
    dblk = NP // C // 10
    return pl.pallas_call(
        _mm_body,
        grid=(N // blk,),
        in_specs=[
            pl.BlockSpec((blk, C), lambda i: (i, 0)),
            pl.BlockSpec((C, DIM), lambda i: (0, 0)),
            pl.BlockSpec((dblk, C), lambda i: (i, 0)),
            pl.BlockSpec((dblk, C), lambda i: (i, 0)),
        ],
        out_specs=[
            pl.BlockSpec((blk, DIM), lambda i: (i, 0)),
            pl.BlockSpec((dblk, C), lambda i: (i, 0)),
        ],
        out_shape=[
            jax.ShapeDtypeStruct((N, DIM), jnp.float32),
            jax.ShapeDtypeStruct((NP // C, C), jnp.float32),
        ],
    )(x, w, deg0, deg1)


def _tail_body(a0_ref, a1_ref, x_ref, wfc_ref, bg_ref, bfc_ref, sb_ref,
               o_ref):
    a = a0_ref[...] + a1_ref[...] + bg_ref[...]
    act = a * jax.nn.sigmoid(a * sb_ref[...]) * jnp.float32(1.0 / 1.1)
    fx = lax.dot_general(act, wfc_ref[...], (((1,), (1,)), ((), ())),
                         preferred_element_type=jnp.float32)
    o_ref[...] = x_ref[...] + fx + bfc_ref[...]


def _tail(agg0, agg1, x, w_fc, b_gcn, b_fc, sb):
    blk = 1000
    return pl.pallas_call(
        _tail_body,
        grid=(N // blk,),
        in_specs=[
            pl.BlockSpec((blk, DIM), lambda i: (i, 0)),
            pl.BlockSpec((blk, DIM), lambda i: (i, 0)),
            pl.BlockSpec((blk, C), lambda i: (i, 0)),
            pl.BlockSpec((C, DIM), lambda i: (0, 0)),
            pl.BlockSpec((1, DIM), lambda i: (0, 0)),
            pl.BlockSpec((1, C), lambda i: (0, 0)),
            pl.BlockSpec((1, DIM), lambda i: (0, 0)),
        ],
        out_specs=pl.BlockSpec((blk, C), lambda i: (i, 0)),
        out_shape=jax.ShapeDtypeStruct((N, C), jnp.float32),
    )(agg0, agg1, x, w_fc, b_gcn, b_fc, sb)


def kernel(x, edge_index, edge_weight, W_gcn, b_gcn, beta, W_fc, b_fc):
    def _chunked(a):
        a3 = a.reshape(G, CH_REAL, K)
        a3 = jnp.pad(a3, ((0, 0), (0, NCH - CH_REAL), (0, 0)))
        return a3.reshape(G * NCH, K)

    src2 = _chunked(edge_index[0])
    dst2 = _chunked(edge_index[1])
    ew2 = _chunked(edge_weight)
    deg0, deg1 = _sc_degree(dst2, ew2)
    h, dinvr = _matmul_dinv(x, W_gcn, deg0.reshape(NP // C, C),
                            deg1.reshape(NP // C, C))
    agg0, agg1 = _sc_aggregate(src2, dst2, ew2, h, dinvr.reshape(NP))
    sb = jnp.broadcast_to(jax.nn.softplus(beta), (1, DIM)).astype(jnp.float32)
    return _tail(agg0, agg1, x, W_fc, b_gcn[None, :], b_fc[None, :], sb)
